# baseline (speedup 1.0000x reference)
import jax
import jax.numpy as jnp
from jax import lax
from jax.experimental import pallas as pl
from jax.experimental.pallas import tpu as pltpu
from jax.experimental.pallas import tpu_sc as plsc

N = 10000
NPAD = 10240
E = 160000
CHUNK = 128
NCHUNK = E // CHUNK
NW = 32
ROWS_PER_SUB = NPAD // 16
ZROWS = 128
TC_BLK = 1024
EP_BLK = 2000


def _full_spec(a):
    nd = a.ndim
    return pl.BlockSpec(a.shape, lambda i, _n=nd: (0,) * _n)


def _row_spec(width, blk=TC_BLK):
    return pl.BlockSpec((blk, width), lambda i: (i, 0))


def _row3_spec(lead, width, blk=TC_BLK):
    return pl.BlockSpec((lead, blk, width), lambda i: (0, i, 0))


def _eproj(e0_p, e0_q, w1me_p, w2me_p, w2ee_p, w1ee_p, w1me_q, w2me_q, w2ee_q, w1ee_q):
    def body(e0p, e0q, m1p, m2p, e2p, e1p, m1q, m2q, e2q, e1q,
             b1p, eBp, eDp, b1q, eBq, eDq):
        for (e0, m1, m2, e2, e1, ob1, oeB, oeD) in (
                (e0p, m1p, m2p, e2p, e1p, b1p, eBp, eDp),
                (e0q, m1q, m2q, e2q, e1q, b1q, eBq, eDq)):
            x = e0[...]
            compB = jnp.dot(m2[...], e1[...], preferred_element_type=jnp.float32)
            compD = jnp.dot(e2[...], e1[...], preferred_element_type=jnp.float32)
            ob1[...] = jnp.dot(x, m1[...].T, preferred_element_type=jnp.float32)
            oeB[...] = jnp.dot(x, compB.T, preferred_element_type=jnp.float32)
            oeD[...] = jnp.dot(x, compD.T, preferred_element_type=jnp.float32)

    outs = [jax.ShapeDtypeStruct((E, 128), jnp.float32)] * 6
    ws = (w1me_p, w2me_p, w2ee_p, w1ee_p, w1me_q, w2me_q, w2ee_q, w1ee_q)
    return pl.pallas_call(
        body,
        grid=(E // EP_BLK,),
        in_specs=[_row_spec(16, EP_BLK), _row_spec(16, EP_BLK)] + [_full_spec(w) for w in ws],
        out_specs=[_row_spec(128, EP_BLK)] * 6,
        out_shape=outs,
    )(e0_p, e0_q, *ws)


def _node_a(px, mx, w1ms_p, b1m_p, w1es_p, b1e_p, w1ms_q, b1m_q, w1es_q, b1e_q):
    def body(pxr, mxr, wmp, bmp, wep, bep, wmq, bmq, weq, beq,
             t1p, t1q, c1p, c1q):
        p = pxr[...]
        m = mxr[...]
        t1p[...] = jnp.dot(p, wmp[...].T, preferred_element_type=jnp.float32) + bmp[...]
        c1p[...] = jnp.dot(p, wep[...].T, preferred_element_type=jnp.float32) + bep[...]
        t1q[...] = jnp.dot(m, wmq[...].T, preferred_element_type=jnp.float32) + bmq[...]
        c1q[...] = jnp.dot(m, weq[...].T, preferred_element_type=jnp.float32) + beq[...]

    ws = (w1ms_p, b1m_p, w1es_p, b1e_p, w1ms_q, b1m_q, w1es_q, b1e_q)
    return pl.pallas_call(
        body,
        grid=(pl.cdiv(N, TC_BLK),),
        in_specs=[_row_spec(128), _row_spec(128)] + [_full_spec(w) for w in ws],
        out_specs=[_row_spec(128)] * 4,
        out_shape=[jax.ShapeDtypeStruct((N, 128), jnp.float32)] * 4,
    )(px, mx, *ws)


def _stage_b(px, mx, s1c_p, s1c_q, counts, c1_p, c1_q, wdict):
    wnames = ('w1ad_p', 'w1an_p', 'b1a_p', 'w1ad_q', 'w1an_q', 'b1a_q',
              'w2ms_p', 'w2me_p', 'b2m_p', 'w1en_p',
              'w2ms_q', 'w2me_q', 'b2m_q', 'w1en_q')
    ws = [wdict[n] for n in wnames]

    def body(pxr, mxr, s1pr, s1qr, ccr, c1pr, c1qr,
             w1ad_p, w1an_p, b1a_p, w1ad_q, w1an_q, b1a_q,
             w2ms_p, w2me_p, b2m_p, w1en_p,
             w2ms_q, w2me_q, b2m_q, w1en_q,
             mh1_o, ph1_o, ts2p_o, td2p_o, ts2q_o, td2q_o,
             hn1p_o, hn1q_o, cntp_o, cntq_o):
        sp = s1pr[0] + s1pr[1]
        sq = s1qr[0] + s1qr[1]
        cnt_p = jnp.maximum(ccr[0, :, 0:1], 1.0)
        cnt_q = jnp.maximum(ccr[1, :, 0:1], 1.0)
        hn1_p = sp / cnt_p
        hn1_q = sq / cnt_q
        mh1 = (jnp.dot(mxr[...], w1ad_p[...].T, preferred_element_type=jnp.float32)
               + jnp.dot(hn1_p, w1an_p[...].T, preferred_element_type=jnp.float32)
               + b1a_p[...])
        ph1 = (jnp.dot(pxr[...], w1ad_q[...].T, preferred_element_type=jnp.float32)
               + jnp.dot(hn1_q, w1an_q[...].T, preferred_element_type=jnp.float32)
               + b1a_q[...])
        compB_p = jnp.dot(w2me_p[...], w1en_p[...], preferred_element_type=jnp.float32)
        compB_q = jnp.dot(w2me_q[...], w1en_q[...], preferred_element_type=jnp.float32)
        mh1_o[...] = mh1
        ph1_o[...] = ph1
        ts2p_o[...] = (jnp.dot(ph1, w2ms_p[...].T, preferred_element_type=jnp.float32)
                       + jnp.dot(c1pr[...], w2me_p[...].T, preferred_element_type=jnp.float32)
                       + b2m_p[...])
        td2p_o[...] = jnp.dot(hn1_p, compB_p.T, preferred_element_type=jnp.float32)
        ts2q_o[...] = (jnp.dot(mh1, w2ms_q[...].T, preferred_element_type=jnp.float32)
                       + jnp.dot(c1qr[...], w2me_q[...].T, preferred_element_type=jnp.float32)
                       + b2m_q[...])
        td2q_o[...] = jnp.dot(hn1_q, compB_q.T, preferred_element_type=jnp.float32)
        hn1p_o[...] = hn1_p
        hn1q_o[...] = hn1_q
        cntp_o[...] = cnt_p
        cntq_o[...] = cnt_q

    o128 = jax.ShapeDtypeStruct((N, 128), jnp.float32)
    o1 = jax.ShapeDtypeStruct((N, 1), jnp.float32)
    return pl.pallas_call(
        body,
        grid=(pl.cdiv(N, TC_BLK),),
        in_specs=[_row_spec(128), _row_spec(128), _row3_spec(2, 128), _row3_spec(2, 128),
                  _row3_spec(2, 128), _row_spec(128), _row_spec(128)]
                 + [_full_spec(w) for w in ws],
        out_specs=[_row_spec(128)] * 8 + [_row_spec(1), _row_spec(1)],
        out_shape=[o128] * 8 + [o1, o1],
    )(px, mx, s1c_p, s1c_q, counts, c1_p, c1_q, *ws)


def _stage_c(s2_p, s2_q, cnt_p, cnt_q, mh1, ph1, hn1_p, hn1_q, c1_p, c1_q, wdict):
    wnames = ('w2ad_p', 'w2an_p', 'b2a_p', 'w2ad_q', 'w2an_q', 'b2a_q',
              'w2es_p', 'w2ee_p', 'b2e_p', 'w2en_p', 'w1en_p',
              'w2es_q', 'w2ee_q', 'b2e_q', 'w2en_q', 'w1en_q')
    ws = [wdict[n] for n in wnames]

    def body(s2pr, s2qr, cntpr, cntqr, mh1r, ph1r, hn1pr, hn1qr, c1pr, c1qr,
             w2ad_p, w2an_p, b2a_p, w2ad_q, w2an_q, b2a_q,
             w2es_p, w2ee_p, b2e_p, w2en_p, w1en_p,
             w2es_q, w2ee_q, b2e_q, w2en_q, w1en_q,
             mout_o, pout_o, usp_o, udp_o, usq_o, udq_o):
        hn2_p = (s2pr[0] + s2pr[1]) / cntpr[...]
        hn2_q = (s2qr[0] + s2qr[1]) / cntqr[...]
        mout_o[...] = (jnp.dot(mh1r[...], w2ad_p[...].T, preferred_element_type=jnp.float32)
                       + jnp.dot(hn2_p, w2an_p[...].T, preferred_element_type=jnp.float32)
                       + b2a_p[...])
        pout_o[...] = (jnp.dot(ph1r[...], w2ad_q[...].T, preferred_element_type=jnp.float32)
                       + jnp.dot(hn2_q, w2an_q[...].T, preferred_element_type=jnp.float32)
                       + b2a_q[...])
        usp_o[...] = (jnp.dot(ph1r[...], w2es_p[...].T, preferred_element_type=jnp.float32)
                      + jnp.dot(c1pr[...], w2ee_p[...].T, preferred_element_type=jnp.float32)
                      + b2e_p[...])
        compE_p = jnp.dot(w2ee_p[...], w1en_p[...], preferred_element_type=jnp.float32)
        compE_q = jnp.dot(w2ee_q[...], w1en_q[...], preferred_element_type=jnp.float32)
        udp_o[...] = (jnp.dot(hn1pr[...], compE_p.T, preferred_element_type=jnp.float32)
                      + jnp.dot(hn2_p, w2en_p[...].T, preferred_element_type=jnp.float32))
        usq_o[...] = (jnp.dot(mh1r[...], w2es_q[...].T, preferred_element_type=jnp.float32)
                      + jnp.dot(c1qr[...], w2ee_q[...].T, preferred_element_type=jnp.float32)
                      + b2e_q[...])
        udq_o[...] = (jnp.dot(hn1qr[...], compE_q.T, preferred_element_type=jnp.float32)
                      + jnp.dot(hn2_q, w2en_q[...].T, preferred_element_type=jnp.float32))

    o128 = jax.ShapeDtypeStruct((N, 128), jnp.float32)
    return pl.pallas_call(
        body,
        grid=(pl.cdiv(N, TC_BLK),),
        in_specs=[_row3_spec(2, 128), _row3_spec(2, 128), _row_spec(1), _row_spec(1)]
                 + [_row_spec(128)] * 6 + [_full_spec(w) for w in ws],
        out_specs=[_row_spec(128)] * 6,
        out_shape=[o128] * 6,
    )(s2_p, s2_q, cnt_p, cnt_q, mh1, ph1, hn1_p, hn1_q, c1_p, c1_q, *ws)


def _sc_pass(two_tables, do_relu, do_scatter, chunk):
    mesh = plsc.VectorSubcoreMesh(core_axis_name="c", subcore_axis_name="s")
    W = 128
    nj = W // 16
    nchunk = E // chunk
    rem = nchunk % NW
    base_trips = nchunk // NW
    if do_scatter:
        out_type = jax.ShapeDtypeStruct((2, NPAD, W), jnp.float32)
    else:
        out_type = jax.ShapeDtypeStruct((E, W), jnp.float32)
    scratch = [
        pltpu.VMEM((chunk,), jnp.int32),
        pltpu.VMEM((chunk,), jnp.int32),
        pltpu.VMEM((chunk, W), jnp.float32),
        pltpu.VMEM((chunk, W), jnp.float32),
        pltpu.SemaphoreType.DMA,
        pltpu.SemaphoreType.DMA,
    ]
    if two_tables:
        scratch.insert(3, pltpu.VMEM((chunk, W), jnp.float32))
    if do_scatter:
        scratch.append(pltpu.VMEM_SHARED((NPAD, W), jnp.float32))

    def body(*refs):
        k = 2 if two_tables else 1
        tabs = refs[:k]
        badd, src_h, dst_h = refs[k:k + 3]
        pos = k + 3
        if do_scatter:
            zeros_h = refs[pos]
            pos += 1
        out = refs[pos]
        pos += 1
        if two_tables:
            src_v, dst_v, a_v, b_v, e_v, sem, sem2 = refs[pos:pos + 7]
            pos += 7
        else:
            src_v, dst_v, a_v, e_v, sem, sem2 = refs[pos:pos + 6]
            pos += 6
            b_v = None
        if do_scatter:
            acc = refs[pos]
        cid = lax.axis_index("c")
        sid = lax.axis_index("s")
        wid = sid * 2 + cid

        if do_scatter:
            for kk in range(ROWS_PER_SUB // ZROWS):
                pltpu.sync_copy(zeros_h, acc.at[pl.ds(sid * ROWS_PER_SUB + kk * ZROWS, ZROWS)])
            plsc.subcore_barrier()

        trips = base_trips + jnp.where(wid < rem, 1, 0)

        def trip(j, carry):
            base = (j * NW + wid) * chunk
            pltpu.sync_copy(src_h.at[pl.ds(base, chunk)], src_v)
            pltpu.sync_copy(dst_h.at[pl.ds(base, chunk)], dst_v)
            pltpu.async_copy(tabs[0].at[src_v], a_v, sem).wait()
            if two_tables:
                pltpu.async_copy(tabs[1].at[dst_v], b_v, sem2).wait()
            pltpu.sync_copy(badd.at[pl.ds(base, chunk)], e_v)

            @plsc.parallel_loop(0, chunk, step=1, unroll=2)
            def crow(r):
                for jj in range(nj):
                    s = pl.ds(jj * 16, 16)
                    x = a_v[r, s] + e_v[r, s]
                    if two_tables:
                        x = x + b_v[r, s]
                    if do_relu:
                        x = jnp.maximum(x, 0.0)
                    a_v[r, s] = x

            if do_scatter:
                pltpu.sync_copy(a_v, acc.at[dst_v], add=True)
            else:
                pltpu.sync_copy(a_v, out.at[pl.ds(base, chunk)])
            return carry
        lax.fori_loop(0, trips, trip, 0)

        if do_scatter:
            plsc.subcore_barrier()
            for kk in range(ROWS_PER_SUB // ZROWS):
                r0 = sid * ROWS_PER_SUB + kk * ZROWS
                pltpu.sync_copy(acc.at[pl.ds(r0, ZROWS)], out.at[cid, pl.ds(r0, ZROWS)])

    return pl.kernel(body, out_type=out_type, mesh=mesh, scratch_types=scratch)


def _sc_counts(dst_both):
    mesh = plsc.VectorSubcoreMesh(core_axis_name="c", subcore_axis_name="s")
    rem = NCHUNK % 16
    base_trips = NCHUNK // 16
    out_type = jax.ShapeDtypeStruct((2, NPAD, 128), jnp.float32)
    scratch = [
        pltpu.VMEM((CHUNK,), jnp.int32),
        pltpu.VMEM((CHUNK, 128), jnp.float32),
        pltpu.VMEM_SHARED((NPAD, 128), jnp.float32),
    ]

    def body(dsts, zeros_h, out, dst_v, ones_v, acc):
        cid = lax.axis_index("c")
        sid = lax.axis_index("s")

        @plsc.parallel_loop(0, CHUNK, step=1, unroll=2)
        def fill(r):
            for j in range(8):
                ones_v[r, pl.ds(j * 16, 16)] = jnp.ones((16,), jnp.float32)
        for kk in range(ROWS_PER_SUB // ZROWS):
            pltpu.sync_copy(zeros_h, acc.at[pl.ds(sid * ROWS_PER_SUB + kk * ZROWS, ZROWS)])
        plsc.subcore_barrier()

        trips = base_trips + jnp.where(sid < rem, 1, 0)

        def trip(j, carry):
            base = (j * 16 + sid) * CHUNK
            pltpu.sync_copy(dsts.at[cid, pl.ds(base, CHUNK)], dst_v)
            pltpu.sync_copy(ones_v, acc.at[dst_v], add=True)
            return carry
        lax.fori_loop(0, trips, trip, 0)

        plsc.subcore_barrier()
        for kk in range(ROWS_PER_SUB // ZROWS):
            r0 = sid * ROWS_PER_SUB + kk * ZROWS
            pltpu.sync_copy(acc.at[pl.ds(r0, ZROWS)], out.at[cid, pl.ds(r0, ZROWS)])

    zeros = jnp.zeros((ZROWS, 128), jnp.float32)
    return pl.kernel(body, out_type=out_type, mesh=mesh, scratch_types=scratch)(dst_both, zeros)


def _p1(table, badd, src, dst):
    zeros = jnp.zeros((ZROWS, 128), jnp.float32)
    return _sc_pass(False, True, True, 80)(table, badd, src, dst, zeros)


def _p2(tsrc, tdst, badd, src, dst):
    zeros = jnp.zeros((ZROWS, 128), jnp.float32)
    return _sc_pass(True, True, True, 80)(tsrc, tdst, badd, src, dst, zeros)


def _p3(tsrc, tdst, badd, src, dst):
    return _sc_pass(True, False, False, 128)(tsrc, tdst, badd, src, dst)


def kernel(player_x, match_x, plays_efeats, played_by_efeats,
           plays_edge_index, played_by_edge_index, params):
    f32 = jnp.float32
    p_src = plays_edge_index[0].astype(jnp.int32)
    p_dst = plays_edge_index[1].astype(jnp.int32)
    q_src = played_by_edge_index[0].astype(jnp.int32)
    q_dst = played_by_edge_index[1].astype(jnp.int32)

    w = {}
    for tag, (c1, c2) in (('p', ('conv1_plays', 'conv2_plays')),
                          ('q', ('conv1_played_by', 'conv2_played_by'))):
        p1, p2 = params[c1], params[c2]
        w1m, b1m = p1['W_msg']['w'], p1['W_msg']['b']
        w1e, b1e = p1['W_edge']['w'], p1['W_edge']['b']
        w1a, b1a = p1['W_apply']['w'], p1['W_apply']['b']
        w2m, b2m = p2['W_msg']['w'], p2['W_msg']['b']
        w2e, b2e = p2['W_edge']['w'], p2['W_edge']['b']
        w2a, b2a = p2['W_apply']['w'], p2['W_apply']['b']
        w['w1ms_' + tag] = w1m[:, :128]
        w['b1m_' + tag] = b1m[None, :]
        w['w1me_' + tag] = w1m[:, 128:144]
        w['w1es_' + tag] = w1e[:, :128]
        w['w1ee_' + tag] = w1e[:, 128:144]
        w['w1en_' + tag] = w1e[:, 144:272]
        w['b1e_' + tag] = b1e[None, :]
        w['w1ad_' + tag] = w1a[:, :128]
        w['w1an_' + tag] = w1a[:, 128:256]
        w['b1a_' + tag] = b1a[None, :]
        w['w2ms_' + tag] = w2m[:, :128]
        w['w2me_' + tag] = w2m[:, 128:256]
        w['b2m_' + tag] = b2m[None, :]
        w['w2es_' + tag] = w2e[:, :128]
        w['w2ee_' + tag] = w2e[:, 128:256]
        w['w2en_' + tag] = w2e[:, 256:384]
        w['b2e_' + tag] = b2e[None, :]
        w['w2ad_' + tag] = w2a[:, :128]
        w['w2an_' + tag] = w2a[:, 128:256]
        w['b2a_' + tag] = b2a[None, :]

    b1_p, eB_p, eD_p, b1_q, eB_q, eD_q = _eproj(
        plays_efeats, played_by_efeats,
        w['w1me_p'], w['w2me_p'], w['w2ee_p'], w['w1ee_p'],
        w['w1me_q'], w['w2me_q'], w['w2ee_q'], w['w1ee_q'])
    t1_p, t1_q, c1_p, c1_q = _node_a(
        player_x, match_x,
        w['w1ms_p'], w['b1m_p'], w['w1es_p'], w['b1e_p'],
        w['w1ms_q'], w['b1m_q'], w['w1es_q'], w['b1e_q'])

    counts = _sc_counts(jnp.stack([p_dst, q_dst]))

    s1c_p = _p1(t1_p, b1_p, p_src, p_dst)
    s1c_q = _p1(t1_q, b1_q, q_src, q_dst)

    (mh1, ph1, ts2_p, td2_p, ts2_q, td2_q,
     hn1_p, hn1_q, cnt_p, cnt_q) = _stage_b(
        player_x, match_x, s1c_p, s1c_q, counts, c1_p, c1_q, w)

    s2_p = _p2(ts2_p, td2_p, eB_p, p_src, p_dst)
    s2_q = _p2(ts2_q, td2_q, eB_q, q_src, q_dst)

    mout, pout, us_p, ud_p, us_q, ud_q = _stage_c(
        s2_p, s2_q, cnt_p, cnt_q, mh1, ph1, hn1_p, hn1_q, c1_p, c1_q, w)

    plays_eh2 = _p3(us_p, ud_p, eD_p, p_src, p_dst)
    pb_eh2 = _p3(us_q, ud_q, eD_q, q_src, q_dst)

    return (mout, pout, plays_eh2, pb_eh2)


# speedup vs baseline: 1.2552x; 1.2552x over previous
import jax
import jax.numpy as jnp
from jax import lax
from jax.experimental import pallas as pl
from jax.experimental.pallas import tpu as pltpu
from jax.experimental.pallas import tpu_sc as plsc

N = 10000
NPAD = 10240
E = 160000
CHUNK = 128
NCHUNK = E // CHUNK
NW = 32
ROWS_PER_SUB = NPAD // 16
ZROWS = 128
TC_BLK = 1024
EP_BLK = 2000


def _full_spec(a):
    nd = a.ndim
    return pl.BlockSpec(a.shape, lambda i, _n=nd: (0,) * _n)


def _row_spec(width, blk=TC_BLK):
    return pl.BlockSpec((blk, width), lambda i: (i, 0))


def _row3_spec(lead, width, blk=TC_BLK):
    return pl.BlockSpec((lead, blk, width), lambda i: (0, i, 0))


def _eproj(e0_p, e0_q, w1me_p, w2me_p, w2ee_p, w1ee_p, w1me_q, w2me_q, w2ee_q, w1ee_q):
    def body(e0p, e0q, m1p, m2p, e2p, e1p, m1q, m2q, e2q, e1q,
             b1p, eBp, eDp, b1q, eBq, eDq):
        for (e0, m1, m2, e2, e1, ob1, oeB, oeD) in (
                (e0p, m1p, m2p, e2p, e1p, b1p, eBp, eDp),
                (e0q, m1q, m2q, e2q, e1q, b1q, eBq, eDq)):
            x = e0[...]
            compB = jnp.dot(m2[...], e1[...], preferred_element_type=jnp.float32)
            compD = jnp.dot(e2[...], e1[...], preferred_element_type=jnp.float32)
            ob1[...] = jnp.dot(x, m1[...].T, preferred_element_type=jnp.float32)
            oeB[...] = jnp.dot(x, compB.T, preferred_element_type=jnp.float32)
            oeD[...] = jnp.dot(x, compD.T, preferred_element_type=jnp.float32)

    outs = [jax.ShapeDtypeStruct((E, 128), jnp.float32)] * 6
    ws = (w1me_p, w2me_p, w2ee_p, w1ee_p, w1me_q, w2me_q, w2ee_q, w1ee_q)
    return pl.pallas_call(
        body,
        grid=(E // EP_BLK,),
        in_specs=[_row_spec(16, EP_BLK), _row_spec(16, EP_BLK)] + [_full_spec(w) for w in ws],
        out_specs=[_row_spec(128, EP_BLK)] * 6,
        out_shape=outs,
    )(e0_p, e0_q, *ws)


def _node_a(px, mx, w1ms_p, b1m_p, w1es_p, b1e_p, w1ms_q, b1m_q, w1es_q, b1e_q):
    def body(pxr, mxr, wmp, bmp, wep, bep, wmq, bmq, weq, beq,
             t1p, t1q, c1p, c1q):
        p = pxr[...]
        m = mxr[...]
        t1p[...] = jnp.dot(p, wmp[...].T, preferred_element_type=jnp.float32) + bmp[...]
        c1p[...] = jnp.dot(p, wep[...].T, preferred_element_type=jnp.float32) + bep[...]
        t1q[...] = jnp.dot(m, wmq[...].T, preferred_element_type=jnp.float32) + bmq[...]
        c1q[...] = jnp.dot(m, weq[...].T, preferred_element_type=jnp.float32) + beq[...]

    ws = (w1ms_p, b1m_p, w1es_p, b1e_p, w1ms_q, b1m_q, w1es_q, b1e_q)
    return pl.pallas_call(
        body,
        grid=(pl.cdiv(N, TC_BLK),),
        in_specs=[_row_spec(128), _row_spec(128)] + [_full_spec(w) for w in ws],
        out_specs=[_row_spec(128)] * 4,
        out_shape=[jax.ShapeDtypeStruct((N, 128), jnp.float32)] * 4,
    )(px, mx, *ws)


def _stage_b(px, mx, s1c_p, s1c_q, counts, c1_p, c1_q, wdict):
    wnames = ('w1ad_p', 'w1an_p', 'b1a_p', 'w1ad_q', 'w1an_q', 'b1a_q',
              'w2ms_p', 'w2me_p', 'b2m_p', 'w1en_p',
              'w2ms_q', 'w2me_q', 'b2m_q', 'w1en_q')
    ws = [wdict[n] for n in wnames]

    def body(pxr, mxr, s1pr, s1qr, ccr, c1pr, c1qr,
             w1ad_p, w1an_p, b1a_p, w1ad_q, w1an_q, b1a_q,
             w2ms_p, w2me_p, b2m_p, w1en_p,
             w2ms_q, w2me_q, b2m_q, w1en_q,
             mh1_o, ph1_o, ts2p_o, td2p_o, ts2q_o, td2q_o,
             hn1p_o, hn1q_o, cntp_o, cntq_o):
        sp = s1pr[0] + s1pr[1]
        sq = s1qr[0] + s1qr[1]
        cnt_p = jnp.maximum(ccr[0, :, 0:1], 1.0)
        cnt_q = jnp.maximum(ccr[1, :, 0:1], 1.0)
        hn1_p = sp / cnt_p
        hn1_q = sq / cnt_q
        mh1 = (jnp.dot(mxr[...], w1ad_p[...].T, preferred_element_type=jnp.float32)
               + jnp.dot(hn1_p, w1an_p[...].T, preferred_element_type=jnp.float32)
               + b1a_p[...])
        ph1 = (jnp.dot(pxr[...], w1ad_q[...].T, preferred_element_type=jnp.float32)
               + jnp.dot(hn1_q, w1an_q[...].T, preferred_element_type=jnp.float32)
               + b1a_q[...])
        compB_p = jnp.dot(w2me_p[...], w1en_p[...], preferred_element_type=jnp.float32)
        compB_q = jnp.dot(w2me_q[...], w1en_q[...], preferred_element_type=jnp.float32)
        mh1_o[...] = mh1
        ph1_o[...] = ph1
        ts2p_o[...] = (jnp.dot(ph1, w2ms_p[...].T, preferred_element_type=jnp.float32)
                       + jnp.dot(c1pr[...], w2me_p[...].T, preferred_element_type=jnp.float32)
                       + b2m_p[...])
        td2p_o[...] = jnp.dot(hn1_p, compB_p.T, preferred_element_type=jnp.float32)
        ts2q_o[...] = (jnp.dot(mh1, w2ms_q[...].T, preferred_element_type=jnp.float32)
                       + jnp.dot(c1qr[...], w2me_q[...].T, preferred_element_type=jnp.float32)
                       + b2m_q[...])
        td2q_o[...] = jnp.dot(hn1_q, compB_q.T, preferred_element_type=jnp.float32)
        hn1p_o[...] = hn1_p
        hn1q_o[...] = hn1_q
        cntp_o[...] = cnt_p
        cntq_o[...] = cnt_q

    o128 = jax.ShapeDtypeStruct((N, 128), jnp.float32)
    o1 = jax.ShapeDtypeStruct((N, 1), jnp.float32)
    return pl.pallas_call(
        body,
        grid=(pl.cdiv(N, TC_BLK),),
        in_specs=[_row_spec(128), _row_spec(128), _row3_spec(2, 128), _row3_spec(2, 128),
                  _row3_spec(2, 128), _row_spec(128), _row_spec(128)]
                 + [_full_spec(w) for w in ws],
        out_specs=[_row_spec(128)] * 8 + [_row_spec(1), _row_spec(1)],
        out_shape=[o128] * 8 + [o1, o1],
    )(px, mx, s1c_p, s1c_q, counts, c1_p, c1_q, *ws)


def _stage_c(s2_p, s2_q, cnt_p, cnt_q, mh1, ph1, hn1_p, hn1_q, c1_p, c1_q, wdict):
    wnames = ('w2ad_p', 'w2an_p', 'b2a_p', 'w2ad_q', 'w2an_q', 'b2a_q',
              'w2es_p', 'w2ee_p', 'b2e_p', 'w2en_p', 'w1en_p',
              'w2es_q', 'w2ee_q', 'b2e_q', 'w2en_q', 'w1en_q')
    ws = [wdict[n] for n in wnames]

    def body(s2pr, s2qr, cntpr, cntqr, mh1r, ph1r, hn1pr, hn1qr, c1pr, c1qr,
             w2ad_p, w2an_p, b2a_p, w2ad_q, w2an_q, b2a_q,
             w2es_p, w2ee_p, b2e_p, w2en_p, w1en_p,
             w2es_q, w2ee_q, b2e_q, w2en_q, w1en_q,
             mout_o, pout_o, usp_o, udp_o, usq_o, udq_o):
        hn2_p = (s2pr[0] + s2pr[1]) / cntpr[...]
        hn2_q = (s2qr[0] + s2qr[1]) / cntqr[...]
        mout_o[...] = (jnp.dot(mh1r[...], w2ad_p[...].T, preferred_element_type=jnp.float32)
                       + jnp.dot(hn2_p, w2an_p[...].T, preferred_element_type=jnp.float32)
                       + b2a_p[...])
        pout_o[...] = (jnp.dot(ph1r[...], w2ad_q[...].T, preferred_element_type=jnp.float32)
                       + jnp.dot(hn2_q, w2an_q[...].T, preferred_element_type=jnp.float32)
                       + b2a_q[...])
        usp_o[...] = (jnp.dot(ph1r[...], w2es_p[...].T, preferred_element_type=jnp.float32)
                      + jnp.dot(c1pr[...], w2ee_p[...].T, preferred_element_type=jnp.float32)
                      + b2e_p[...])
        compE_p = jnp.dot(w2ee_p[...], w1en_p[...], preferred_element_type=jnp.float32)
        compE_q = jnp.dot(w2ee_q[...], w1en_q[...], preferred_element_type=jnp.float32)
        udp_o[...] = (jnp.dot(hn1pr[...], compE_p.T, preferred_element_type=jnp.float32)
                      + jnp.dot(hn2_p, w2en_p[...].T, preferred_element_type=jnp.float32))
        usq_o[...] = (jnp.dot(mh1r[...], w2es_q[...].T, preferred_element_type=jnp.float32)
                      + jnp.dot(c1qr[...], w2ee_q[...].T, preferred_element_type=jnp.float32)
                      + b2e_q[...])
        udq_o[...] = (jnp.dot(hn1qr[...], compE_q.T, preferred_element_type=jnp.float32)
                      + jnp.dot(hn2_q, w2en_q[...].T, preferred_element_type=jnp.float32))

    o128 = jax.ShapeDtypeStruct((N, 128), jnp.float32)
    return pl.pallas_call(
        body,
        grid=(pl.cdiv(N, TC_BLK),),
        in_specs=[_row3_spec(2, 128), _row3_spec(2, 128), _row_spec(1), _row_spec(1)]
                 + [_row_spec(128)] * 6 + [_full_spec(w) for w in ws],
        out_specs=[_row_spec(128)] * 6,
        out_shape=[o128] * 6,
    )(s2_p, s2_q, cnt_p, cnt_q, mh1, ph1, hn1_p, hn1_q, c1_p, c1_q, *ws)


def _sc_pass(two_tables, do_relu, do_scatter, chunk):
    mesh = plsc.VectorSubcoreMesh(core_axis_name="c", subcore_axis_name="s")
    W = 128
    nj = W // 16
    nchunk = E // chunk
    rem = nchunk % NW
    base_trips = nchunk // NW
    if do_scatter:
        out_type = jax.ShapeDtypeStruct((2, NPAD, W), jnp.float32)
    else:
        out_type = jax.ShapeDtypeStruct((E, W), jnp.float32)
    scratch = [
        pltpu.VMEM((chunk,), jnp.int32),
        pltpu.VMEM((chunk,), jnp.int32),
        pltpu.VMEM((chunk, W), jnp.float32),
        pltpu.VMEM((chunk, W), jnp.float32),
        pltpu.SemaphoreType.DMA,
        pltpu.SemaphoreType.DMA,
        pltpu.SemaphoreType.DMA,
        pltpu.SemaphoreType.DMA,
        pltpu.SemaphoreType.DMA,
    ]
    if two_tables:
        scratch.insert(3, pltpu.VMEM((chunk, W), jnp.float32))
    if do_scatter:
        scratch.append(pltpu.VMEM_SHARED((NPAD, W), jnp.float32))

    def body(*refs):
        k = 2 if two_tables else 1
        tabs = refs[:k]
        badd, src_h, dst_h = refs[k:k + 3]
        pos = k + 3
        if do_scatter:
            zeros_h = refs[pos]
            pos += 1
        out = refs[pos]
        pos += 1
        if two_tables:
            src_v, dst_v, a_v, b_v, e_v = refs[pos:pos + 5]
            pos += 5
        else:
            src_v, dst_v, a_v, e_v = refs[pos:pos + 4]
            pos += 4
            b_v = None
        sem, sem2, sem3, sem4, sem5 = refs[pos:pos + 5]
        pos += 5
        if do_scatter:
            acc = refs[pos]
        cid = lax.axis_index("c")
        sid = lax.axis_index("s")
        wid = sid * 2 + cid

        if do_scatter:
            for kk in range(ROWS_PER_SUB // ZROWS):
                pltpu.sync_copy(zeros_h, acc.at[pl.ds(sid * ROWS_PER_SUB + kk * ZROWS, ZROWS)])
            plsc.subcore_barrier()

        trips = base_trips + jnp.where(wid < rem, 1, 0)

        def trip(j, carry):
            base = (j * NW + wid) * chunk
            ci1 = pltpu.async_copy(src_h.at[pl.ds(base, chunk)], src_v, sem4)
            ci2 = pltpu.async_copy(dst_h.at[pl.ds(base, chunk)], dst_v, sem5)
            ci1.wait()
            ci2.wait()
            ca = pltpu.async_copy(tabs[0].at[src_v], a_v, sem)
            cb = pltpu.async_copy(tabs[1].at[dst_v], b_v, sem2) if two_tables else None
            ce = pltpu.async_copy(badd.at[pl.ds(base, chunk)], e_v, sem3)
            ca.wait()
            if cb is not None:
                cb.wait()
            ce.wait()

            @plsc.parallel_loop(0, chunk, step=1, unroll=2)
            def crow(r):
                for jj in range(nj):
                    s = pl.ds(jj * 16, 16)
                    x = a_v[r, s] + e_v[r, s]
                    if two_tables:
                        x = x + b_v[r, s]
                    if do_relu:
                        x = jnp.maximum(x, 0.0)
                    a_v[r, s] = x

            if do_scatter:
                pltpu.sync_copy(a_v, acc.at[dst_v], add=True)
            else:
                pltpu.sync_copy(a_v, out.at[pl.ds(base, chunk)])
            return carry
        lax.fori_loop(0, trips, trip, 0)

        if do_scatter:
            plsc.subcore_barrier()
            for kk in range(ROWS_PER_SUB // ZROWS):
                r0 = sid * ROWS_PER_SUB + kk * ZROWS
                pltpu.sync_copy(acc.at[pl.ds(r0, ZROWS)], out.at[cid, pl.ds(r0, ZROWS)])

    return pl.kernel(body, out_type=out_type, mesh=mesh, scratch_types=scratch)


def _sc_counts(dst_both):
    mesh = plsc.VectorSubcoreMesh(core_axis_name="c", subcore_axis_name="s")
    rem = NCHUNK % 16
    base_trips = NCHUNK // 16
    out_type = jax.ShapeDtypeStruct((2, NPAD, 128), jnp.float32)
    scratch = [
        pltpu.VMEM((CHUNK,), jnp.int32),
        pltpu.VMEM((CHUNK, 128), jnp.float32),
        pltpu.VMEM_SHARED((NPAD, 128), jnp.float32),
    ]

    def body(dsts, zeros_h, out, dst_v, ones_v, acc):
        cid = lax.axis_index("c")
        sid = lax.axis_index("s")

        @plsc.parallel_loop(0, CHUNK, step=1, unroll=2)
        def fill(r):
            for j in range(8):
                ones_v[r, pl.ds(j * 16, 16)] = jnp.ones((16,), jnp.float32)
        for kk in range(ROWS_PER_SUB // ZROWS):
            pltpu.sync_copy(zeros_h, acc.at[pl.ds(sid * ROWS_PER_SUB + kk * ZROWS, ZROWS)])
        plsc.subcore_barrier()

        trips = base_trips + jnp.where(sid < rem, 1, 0)

        def trip(j, carry):
            base = (j * 16 + sid) * CHUNK
            pltpu.sync_copy(dsts.at[cid, pl.ds(base, CHUNK)], dst_v)
            pltpu.sync_copy(ones_v, acc.at[dst_v], add=True)
            return carry
        lax.fori_loop(0, trips, trip, 0)

        plsc.subcore_barrier()
        for kk in range(ROWS_PER_SUB // ZROWS):
            r0 = sid * ROWS_PER_SUB + kk * ZROWS
            pltpu.sync_copy(acc.at[pl.ds(r0, ZROWS)], out.at[cid, pl.ds(r0, ZROWS)])

    zeros = jnp.zeros((ZROWS, 128), jnp.float32)
    return pl.kernel(body, out_type=out_type, mesh=mesh, scratch_types=scratch)(dst_both, zeros)


def _p1(table, badd, src, dst):
    zeros = jnp.zeros((ZROWS, 128), jnp.float32)
    return _sc_pass(False, True, True, 80)(table, badd, src, dst, zeros)


def _p2(tsrc, tdst, badd, src, dst):
    zeros = jnp.zeros((ZROWS, 128), jnp.float32)
    return _sc_pass(True, True, True, 80)(tsrc, tdst, badd, src, dst, zeros)


def _p3(tsrc, tdst, badd, src, dst):
    return _sc_pass(True, False, False, 128)(tsrc, tdst, badd, src, dst)


def kernel(player_x, match_x, plays_efeats, played_by_efeats,
           plays_edge_index, played_by_edge_index, params):
    f32 = jnp.float32
    p_src = plays_edge_index[0].astype(jnp.int32)
    p_dst = plays_edge_index[1].astype(jnp.int32)
    q_src = played_by_edge_index[0].astype(jnp.int32)
    q_dst = played_by_edge_index[1].astype(jnp.int32)

    w = {}
    for tag, (c1, c2) in (('p', ('conv1_plays', 'conv2_plays')),
                          ('q', ('conv1_played_by', 'conv2_played_by'))):
        p1, p2 = params[c1], params[c2]
        w1m, b1m = p1['W_msg']['w'], p1['W_msg']['b']
        w1e, b1e = p1['W_edge']['w'], p1['W_edge']['b']
        w1a, b1a = p1['W_apply']['w'], p1['W_apply']['b']
        w2m, b2m = p2['W_msg']['w'], p2['W_msg']['b']
        w2e, b2e = p2['W_edge']['w'], p2['W_edge']['b']
        w2a, b2a = p2['W_apply']['w'], p2['W_apply']['b']
        w['w1ms_' + tag] = w1m[:, :128]
        w['b1m_' + tag] = b1m[None, :]
        w['w1me_' + tag] = w1m[:, 128:144]
        w['w1es_' + tag] = w1e[:, :128]
        w['w1ee_' + tag] = w1e[:, 128:144]
        w['w1en_' + tag] = w1e[:, 144:272]
        w['b1e_' + tag] = b1e[None, :]
        w['w1ad_' + tag] = w1a[:, :128]
        w['w1an_' + tag] = w1a[:, 128:256]
        w['b1a_' + tag] = b1a[None, :]
        w['w2ms_' + tag] = w2m[:, :128]
        w['w2me_' + tag] = w2m[:, 128:256]
        w['b2m_' + tag] = b2m[None, :]
        w['w2es_' + tag] = w2e[:, :128]
        w['w2ee_' + tag] = w2e[:, 128:256]
        w['w2en_' + tag] = w2e[:, 256:384]
        w['b2e_' + tag] = b2e[None, :]
        w['w2ad_' + tag] = w2a[:, :128]
        w['w2an_' + tag] = w2a[:, 128:256]
        w['b2a_' + tag] = b2a[None, :]

    b1_p, eB_p, eD_p, b1_q, eB_q, eD_q = _eproj(
        plays_efeats, played_by_efeats,
        w['w1me_p'], w['w2me_p'], w['w2ee_p'], w['w1ee_p'],
        w['w1me_q'], w['w2me_q'], w['w2ee_q'], w['w1ee_q'])
    t1_p, t1_q, c1_p, c1_q = _node_a(
        player_x, match_x,
        w['w1ms_p'], w['b1m_p'], w['w1es_p'], w['b1e_p'],
        w['w1ms_q'], w['b1m_q'], w['w1es_q'], w['b1e_q'])

    counts = _sc_counts(jnp.stack([p_dst, q_dst]))

    s1c_p = _p1(t1_p, b1_p, p_src, p_dst)
    s1c_q = _p1(t1_q, b1_q, q_src, q_dst)

    (mh1, ph1, ts2_p, td2_p, ts2_q, td2_q,
     hn1_p, hn1_q, cnt_p, cnt_q) = _stage_b(
        player_x, match_x, s1c_p, s1c_q, counts, c1_p, c1_q, w)

    s2_p = _p2(ts2_p, td2_p, eB_p, p_src, p_dst)
    s2_q = _p2(ts2_q, td2_q, eB_q, q_src, q_dst)

    mout, pout, us_p, ud_p, us_q, ud_q = _stage_c(
        s2_p, s2_q, cnt_p, cnt_q, mh1, ph1, hn1_p, hn1_q, c1_p, c1_q, w)

    plays_eh2 = _p3(us_p, ud_p, eD_p, p_src, p_dst)
    pb_eh2 = _p3(us_q, ud_q, eD_q, q_src, q_dst)

    return (mout, pout, plays_eh2, pb_eh2)


# speedup vs baseline: 1.6014x; 1.2758x over previous
import jax
import jax.numpy as jnp
from jax import lax
from jax.experimental import pallas as pl
from jax.experimental.pallas import tpu as pltpu
from jax.experimental.pallas import tpu_sc as plsc

N = 10000
NPAD = 10240
E = 160000
CHUNK = 128
NCHUNK = E // CHUNK
NW = 32
ROWS_PER_SUB = NPAD // 16
ZROWS = 128
TC_BLK = 1024
EP_BLK = 2000


def _full_spec(a):
    nd = a.ndim
    return pl.BlockSpec(a.shape, lambda i, _n=nd: (0,) * _n)


def _row_spec(width, blk=TC_BLK):
    return pl.BlockSpec((blk, width), lambda i: (i, 0))


def _row3_spec(lead, width, blk=TC_BLK):
    return pl.BlockSpec((lead, blk, width), lambda i: (0, i, 0))


def _eproj(e0_p, e0_q, w1me_p, w2me_p, w2ee_p, w1ee_p, w1me_q, w2me_q, w2ee_q, w1ee_q):
    def body(e0p, e0q, m1p, m2p, e2p, e1p, m1q, m2q, e2q, e1q,
             b1p, eBp, eDp, b1q, eBq, eDq):
        for (e0, m1, m2, e2, e1, ob1, oeB, oeD) in (
                (e0p, m1p, m2p, e2p, e1p, b1p, eBp, eDp),
                (e0q, m1q, m2q, e2q, e1q, b1q, eBq, eDq)):
            x = e0[...]
            compB = jnp.dot(m2[...], e1[...], preferred_element_type=jnp.float32)
            compD = jnp.dot(e2[...], e1[...], preferred_element_type=jnp.float32)
            ob1[...] = jnp.dot(x, m1[...].T, preferred_element_type=jnp.float32)
            oeB[...] = jnp.dot(x, compB.T, preferred_element_type=jnp.float32)
            oeD[...] = jnp.dot(x, compD.T, preferred_element_type=jnp.float32)

    outs = [jax.ShapeDtypeStruct((E, 128), jnp.float32)] * 6
    ws = (w1me_p, w2me_p, w2ee_p, w1ee_p, w1me_q, w2me_q, w2ee_q, w1ee_q)
    return pl.pallas_call(
        body,
        grid=(E // EP_BLK,),
        in_specs=[_row_spec(16, EP_BLK), _row_spec(16, EP_BLK)] + [_full_spec(w) for w in ws],
        out_specs=[_row_spec(128, EP_BLK)] * 6,
        out_shape=outs,
    )(e0_p, e0_q, *ws)


def _node_a(px, mx, w1ms_p, b1m_p, w1es_p, b1e_p, w1ms_q, b1m_q, w1es_q, b1e_q):
    def body(pxr, mxr, wmp, bmp, wep, bep, wmq, bmq, weq, beq,
             t1p, t1q, c1p, c1q):
        p = pxr[...]
        m = mxr[...]
        t1p[...] = jnp.dot(p, wmp[...].T, preferred_element_type=jnp.float32) + bmp[...]
        c1p[...] = jnp.dot(p, wep[...].T, preferred_element_type=jnp.float32) + bep[...]
        t1q[...] = jnp.dot(m, wmq[...].T, preferred_element_type=jnp.float32) + bmq[...]
        c1q[...] = jnp.dot(m, weq[...].T, preferred_element_type=jnp.float32) + beq[...]

    ws = (w1ms_p, b1m_p, w1es_p, b1e_p, w1ms_q, b1m_q, w1es_q, b1e_q)
    return pl.pallas_call(
        body,
        grid=(pl.cdiv(N, TC_BLK),),
        in_specs=[_row_spec(128), _row_spec(128)] + [_full_spec(w) for w in ws],
        out_specs=[_row_spec(128)] * 4,
        out_shape=[jax.ShapeDtypeStruct((N, 128), jnp.float32)] * 4,
    )(px, mx, *ws)


def _stage_b(px, mx, s1c_p, s1c_q, counts, c1_p, c1_q, wdict):
    wnames = ('w1ad_p', 'w1an_p', 'b1a_p', 'w1ad_q', 'w1an_q', 'b1a_q',
              'w2ms_p', 'w2me_p', 'b2m_p', 'w1en_p',
              'w2ms_q', 'w2me_q', 'b2m_q', 'w1en_q')
    ws = [wdict[n] for n in wnames]

    def body(pxr, mxr, s1pr, s1qr, ccr, c1pr, c1qr,
             w1ad_p, w1an_p, b1a_p, w1ad_q, w1an_q, b1a_q,
             w2ms_p, w2me_p, b2m_p, w1en_p,
             w2ms_q, w2me_q, b2m_q, w1en_q,
             mh1_o, ph1_o, ts2p_o, td2p_o, ts2q_o, td2q_o,
             hn1p_o, hn1q_o, cntp_o, cntq_o):
        sp = s1pr[0] + s1pr[1]
        sq = s1qr[0] + s1qr[1]
        cnt_p = jnp.maximum(ccr[0, :, 0:1], 1.0)
        cnt_q = jnp.maximum(ccr[1, :, 0:1], 1.0)
        hn1_p = sp / cnt_p
        hn1_q = sq / cnt_q
        mh1 = (jnp.dot(mxr[...], w1ad_p[...].T, preferred_element_type=jnp.float32)
               + jnp.dot(hn1_p, w1an_p[...].T, preferred_element_type=jnp.float32)
               + b1a_p[...])
        ph1 = (jnp.dot(pxr[...], w1ad_q[...].T, preferred_element_type=jnp.float32)
               + jnp.dot(hn1_q, w1an_q[...].T, preferred_element_type=jnp.float32)
               + b1a_q[...])
        compB_p = jnp.dot(w2me_p[...], w1en_p[...], preferred_element_type=jnp.float32)
        compB_q = jnp.dot(w2me_q[...], w1en_q[...], preferred_element_type=jnp.float32)
        mh1_o[...] = mh1
        ph1_o[...] = ph1
        ts2p_o[...] = (jnp.dot(ph1, w2ms_p[...].T, preferred_element_type=jnp.float32)
                       + jnp.dot(c1pr[...], w2me_p[...].T, preferred_element_type=jnp.float32)
                       + b2m_p[...])
        td2p_o[...] = jnp.dot(hn1_p, compB_p.T, preferred_element_type=jnp.float32)
        ts2q_o[...] = (jnp.dot(mh1, w2ms_q[...].T, preferred_element_type=jnp.float32)
                       + jnp.dot(c1qr[...], w2me_q[...].T, preferred_element_type=jnp.float32)
                       + b2m_q[...])
        td2q_o[...] = jnp.dot(hn1_q, compB_q.T, preferred_element_type=jnp.float32)
        hn1p_o[...] = hn1_p
        hn1q_o[...] = hn1_q
        cntp_o[...] = cnt_p
        cntq_o[...] = cnt_q

    o128 = jax.ShapeDtypeStruct((N, 128), jnp.float32)
    o1 = jax.ShapeDtypeStruct((N, 1), jnp.float32)
    return pl.pallas_call(
        body,
        grid=(pl.cdiv(N, TC_BLK),),
        in_specs=[_row_spec(128), _row_spec(128), _row3_spec(2, 128), _row3_spec(2, 128),
                  _row3_spec(2, 128), _row_spec(128), _row_spec(128)]
                 + [_full_spec(w) for w in ws],
        out_specs=[_row_spec(128)] * 8 + [_row_spec(1), _row_spec(1)],
        out_shape=[o128] * 8 + [o1, o1],
    )(px, mx, s1c_p, s1c_q, counts, c1_p, c1_q, *ws)


def _stage_c(s2_p, s2_q, cnt_p, cnt_q, mh1, ph1, hn1_p, hn1_q, c1_p, c1_q, wdict):
    wnames = ('w2ad_p', 'w2an_p', 'b2a_p', 'w2ad_q', 'w2an_q', 'b2a_q',
              'w2es_p', 'w2ee_p', 'b2e_p', 'w2en_p', 'w1en_p',
              'w2es_q', 'w2ee_q', 'b2e_q', 'w2en_q', 'w1en_q')
    ws = [wdict[n] for n in wnames]

    def body(s2pr, s2qr, cntpr, cntqr, mh1r, ph1r, hn1pr, hn1qr, c1pr, c1qr,
             w2ad_p, w2an_p, b2a_p, w2ad_q, w2an_q, b2a_q,
             w2es_p, w2ee_p, b2e_p, w2en_p, w1en_p,
             w2es_q, w2ee_q, b2e_q, w2en_q, w1en_q,
             mout_o, pout_o, usp_o, udp_o, usq_o, udq_o):
        hn2_p = (s2pr[0] + s2pr[1]) / cntpr[...]
        hn2_q = (s2qr[0] + s2qr[1]) / cntqr[...]
        mout_o[...] = (jnp.dot(mh1r[...], w2ad_p[...].T, preferred_element_type=jnp.float32)
                       + jnp.dot(hn2_p, w2an_p[...].T, preferred_element_type=jnp.float32)
                       + b2a_p[...])
        pout_o[...] = (jnp.dot(ph1r[...], w2ad_q[...].T, preferred_element_type=jnp.float32)
                       + jnp.dot(hn2_q, w2an_q[...].T, preferred_element_type=jnp.float32)
                       + b2a_q[...])
        usp_o[...] = (jnp.dot(ph1r[...], w2es_p[...].T, preferred_element_type=jnp.float32)
                      + jnp.dot(c1pr[...], w2ee_p[...].T, preferred_element_type=jnp.float32)
                      + b2e_p[...])
        compE_p = jnp.dot(w2ee_p[...], w1en_p[...], preferred_element_type=jnp.float32)
        compE_q = jnp.dot(w2ee_q[...], w1en_q[...], preferred_element_type=jnp.float32)
        udp_o[...] = (jnp.dot(hn1pr[...], compE_p.T, preferred_element_type=jnp.float32)
                      + jnp.dot(hn2_p, w2en_p[...].T, preferred_element_type=jnp.float32))
        usq_o[...] = (jnp.dot(mh1r[...], w2es_q[...].T, preferred_element_type=jnp.float32)
                      + jnp.dot(c1qr[...], w2ee_q[...].T, preferred_element_type=jnp.float32)
                      + b2e_q[...])
        udq_o[...] = (jnp.dot(hn1qr[...], compE_q.T, preferred_element_type=jnp.float32)
                      + jnp.dot(hn2_q, w2en_q[...].T, preferred_element_type=jnp.float32))

    o128 = jax.ShapeDtypeStruct((N, 128), jnp.float32)
    return pl.pallas_call(
        body,
        grid=(pl.cdiv(N, TC_BLK),),
        in_specs=[_row3_spec(2, 128), _row3_spec(2, 128), _row_spec(1), _row_spec(1)]
                 + [_row_spec(128)] * 6 + [_full_spec(w) for w in ws],
        out_specs=[_row_spec(128)] * 6,
        out_shape=[o128] * 6,
    )(s2_p, s2_q, cnt_p, cnt_q, mh1, ph1, hn1_p, hn1_q, c1_p, c1_q, *ws)


def _sc_pass(two_tables, do_relu, do_scatter, chunk):
    mesh = plsc.VectorSubcoreMesh(core_axis_name="c", subcore_axis_name="s")
    W = 128
    nj = W // 16
    T = E // (chunk * NW)
    assert E == T * chunk * NW and chunk % 8 == 0
    if do_scatter:
        out_type = jax.ShapeDtypeStruct((2, NPAD, W), jnp.float32)
    else:
        out_type = jax.ShapeDtypeStruct((E, W), jnp.float32)
    assert T % 4 == 1
    nbuf_data = 6 if two_tables else 4
    scratch = (
        [pltpu.VMEM((chunk,), jnp.int32) for _ in range(8)]
        + [pltpu.VMEM((chunk, W), jnp.float32) for _ in range(nbuf_data)]
        + [pltpu.SemaphoreType.DMA for _ in range(16)]
    )
    if do_scatter:
        scratch.append(pltpu.VMEM_SHARED((NPAD, W), jnp.float32))

    def body(*refs):
        nt = 2 if two_tables else 1
        tabs = refs[:nt]
        badd, src_h, dst_h = refs[nt:nt + 3]
        pos = nt + 3
        if do_scatter:
            zeros_h = refs[pos]
            pos += 1
        out = refs[pos]
        pos += 1
        src_v = refs[pos:pos + 4]
        dst_v = refs[pos + 4:pos + 8]
        pos += 8
        a_v = refs[pos:pos + 2]
        pos += 2
        if two_tables:
            b_v = refs[pos:pos + 2]
            pos += 2
        else:
            b_v = None
        e_v = refs[pos:pos + 2]
        pos += 2
        sem_is = refs[pos:pos + 4]
        sem_id = refs[pos + 4:pos + 8]
        sem_a = refs[pos + 8:pos + 10]
        sem_b = refs[pos + 10:pos + 12]
        sem_e = refs[pos + 12:pos + 14]
        sem_s = refs[pos + 14:pos + 16]
        pos += 16
        if do_scatter:
            acc = refs[pos]
        cid = lax.axis_index("c")
        sid = lax.axis_index("s")
        wid = sid * 2 + cid

        if do_scatter:
            for kk in range(ROWS_PER_SUB // ZROWS):
                pltpu.sync_copy(zeros_h, acc.at[pl.ds(sid * ROWS_PER_SUB + kk * ZROWS, ZROWS)])
            plsc.subcore_barrier()

        def ebase(t):
            return (t * NW + wid) * chunk

        def idx_desc(p, t):
            b = ebase(t)
            return (pltpu.make_async_copy(src_h.at[pl.ds(b, chunk)], src_v[p], sem_is[p]),
                    pltpu.make_async_copy(dst_h.at[pl.ds(b, chunk)], dst_v[p], sem_id[p]))

        def idx_issue(p, t):
            b = ebase(t)
            pltpu.async_copy(src_h.at[pl.ds(b, chunk)], src_v[p], sem_is[p])
            pltpu.async_copy(dst_h.at[pl.ds(b, chunk)], dst_v[p], sem_id[p])

        def gather_desc(k, p, t):
            ds_ = [pltpu.make_async_copy(tabs[0].at[src_v[p]], a_v[k], sem_a[k]),
                   pltpu.make_async_copy(badd.at[pl.ds(ebase(t), chunk)], e_v[k], sem_e[k])]
            if two_tables:
                ds_.append(pltpu.make_async_copy(tabs[1].at[dst_v[p]], b_v[k], sem_b[k]))
            return ds_

        def gather_issue(k, p, t):
            pltpu.async_copy(tabs[0].at[src_v[p]], a_v[k], sem_a[k])
            pltpu.async_copy(badd.at[pl.ds(ebase(t), chunk)], e_v[k], sem_e[k])
            if two_tables:
                pltpu.async_copy(tabs[1].at[dst_v[p]], b_v[k], sem_b[k])

        def store_sync(k, p, t):
            if do_scatter:
                pltpu.async_copy(a_v[k], acc.at[dst_v[p]], sem_s[k]).wait()
            else:
                pltpu.async_copy(a_v[k], out.at[pl.ds(ebase(t), chunk)], sem_s[k]).wait()

        def compute(k):
            @plsc.parallel_loop(0, chunk, step=1, unroll=2)
            def crow(r):
                for jj in range(nj):
                    s = pl.ds(jj * 16, 16)
                    x = a_v[k][r, s] + e_v[k][r, s]
                    if two_tables:
                        x = x + b_v[k][r, s]
                    if do_relu:
                        x = jnp.maximum(x, 0.0)
                    a_v[k][r, s] = x

        def phase(p, t):
            k = p % 2
            k1 = 1 - k
            pn = (p + 1) % 4
            pp = (p + 2) % 4
            p_prev = (p + 3) % 4
            for d in gather_desc(k, p, t):
                d.wait()

            @pl.when(t + 2 < T)
            def _():
                idx_issue(pp, t + 2)

            @pl.when(t + 1 < T)
            def _():
                for d in idx_desc(pn, t + 1):
                    d.wait()
                gather_issue(k1, pn, t + 1)

            compute(k)
            store_sync(k, p, t)

        idx_issue(0, 0)
        idx_issue(1, 1)
        for d in idx_desc(0, 0):
            d.wait()
        gather_issue(0, 0, 0)

        def quad(m, carry):
            phase(0, 4 * m)
            phase(1, 4 * m + 1)
            phase(2, 4 * m + 2)
            phase(3, 4 * m + 3)
            return carry
        lax.fori_loop(0, (T - 1) // 4, quad, 0)
        phase(0, T - 1)

        if do_scatter:
            plsc.subcore_barrier()
            for kk in range(ROWS_PER_SUB // ZROWS):
                r0 = sid * ROWS_PER_SUB + kk * ZROWS
                pltpu.sync_copy(acc.at[pl.ds(r0, ZROWS)], out.at[cid, pl.ds(r0, ZROWS)])

    return pl.kernel(body, out_type=out_type, mesh=mesh, scratch_types=scratch)


def _sc_counts(dst_both):
    mesh = plsc.VectorSubcoreMesh(core_axis_name="c", subcore_axis_name="s")
    rem = NCHUNK % 16
    base_trips = NCHUNK // 16
    out_type = jax.ShapeDtypeStruct((2, NPAD, 128), jnp.float32)
    scratch = [
        pltpu.VMEM((CHUNK,), jnp.int32),
        pltpu.VMEM((CHUNK, 128), jnp.float32),
        pltpu.VMEM_SHARED((NPAD, 128), jnp.float32),
    ]

    def body(dsts, zeros_h, out, dst_v, ones_v, acc):
        cid = lax.axis_index("c")
        sid = lax.axis_index("s")

        @plsc.parallel_loop(0, CHUNK, step=1, unroll=2)
        def fill(r):
            for j in range(8):
                ones_v[r, pl.ds(j * 16, 16)] = jnp.ones((16,), jnp.float32)
        for kk in range(ROWS_PER_SUB // ZROWS):
            pltpu.sync_copy(zeros_h, acc.at[pl.ds(sid * ROWS_PER_SUB + kk * ZROWS, ZROWS)])
        plsc.subcore_barrier()

        trips = base_trips + jnp.where(sid < rem, 1, 0)

        def trip(j, carry):
            base = (j * 16 + sid) * CHUNK
            pltpu.sync_copy(dsts.at[cid, pl.ds(base, CHUNK)], dst_v)
            pltpu.sync_copy(ones_v, acc.at[dst_v], add=True)
            return carry
        lax.fori_loop(0, trips, trip, 0)

        plsc.subcore_barrier()
        for kk in range(ROWS_PER_SUB // ZROWS):
            r0 = sid * ROWS_PER_SUB + kk * ZROWS
            pltpu.sync_copy(acc.at[pl.ds(r0, ZROWS)], out.at[cid, pl.ds(r0, ZROWS)])

    zeros = jnp.zeros((ZROWS, 128), jnp.float32)
    return pl.kernel(body, out_type=out_type, mesh=mesh, scratch_types=scratch)(dst_both, zeros)


def _p1(table, badd, src, dst):
    zeros = jnp.zeros((ZROWS, 128), jnp.float32)
    return _sc_pass(False, True, True, 40)(table, badd, src, dst, zeros)


def _p2(tsrc, tdst, badd, src, dst):
    zeros = jnp.zeros((ZROWS, 128), jnp.float32)
    return _sc_pass(True, True, True, 40)(tsrc, tdst, badd, src, dst, zeros)


def _p3(tsrc, tdst, badd, src, dst):
    return _sc_pass(True, False, False, 40)(tsrc, tdst, badd, src, dst)


def kernel(player_x, match_x, plays_efeats, played_by_efeats,
           plays_edge_index, played_by_edge_index, params):
    f32 = jnp.float32
    p_src = plays_edge_index[0].astype(jnp.int32)
    p_dst = plays_edge_index[1].astype(jnp.int32)
    q_src = played_by_edge_index[0].astype(jnp.int32)
    q_dst = played_by_edge_index[1].astype(jnp.int32)

    w = {}
    for tag, (c1, c2) in (('p', ('conv1_plays', 'conv2_plays')),
                          ('q', ('conv1_played_by', 'conv2_played_by'))):
        p1, p2 = params[c1], params[c2]
        w1m, b1m = p1['W_msg']['w'], p1['W_msg']['b']
        w1e, b1e = p1['W_edge']['w'], p1['W_edge']['b']
        w1a, b1a = p1['W_apply']['w'], p1['W_apply']['b']
        w2m, b2m = p2['W_msg']['w'], p2['W_msg']['b']
        w2e, b2e = p2['W_edge']['w'], p2['W_edge']['b']
        w2a, b2a = p2['W_apply']['w'], p2['W_apply']['b']
        w['w1ms_' + tag] = w1m[:, :128]
        w['b1m_' + tag] = b1m[None, :]
        w['w1me_' + tag] = w1m[:, 128:144]
        w['w1es_' + tag] = w1e[:, :128]
        w['w1ee_' + tag] = w1e[:, 128:144]
        w['w1en_' + tag] = w1e[:, 144:272]
        w['b1e_' + tag] = b1e[None, :]
        w['w1ad_' + tag] = w1a[:, :128]
        w['w1an_' + tag] = w1a[:, 128:256]
        w['b1a_' + tag] = b1a[None, :]
        w['w2ms_' + tag] = w2m[:, :128]
        w['w2me_' + tag] = w2m[:, 128:256]
        w['b2m_' + tag] = b2m[None, :]
        w['w2es_' + tag] = w2e[:, :128]
        w['w2ee_' + tag] = w2e[:, 128:256]
        w['w2en_' + tag] = w2e[:, 256:384]
        w['b2e_' + tag] = b2e[None, :]
        w['w2ad_' + tag] = w2a[:, :128]
        w['w2an_' + tag] = w2a[:, 128:256]
        w['b2a_' + tag] = b2a[None, :]

    b1_p, eB_p, eD_p, b1_q, eB_q, eD_q = _eproj(
        plays_efeats, played_by_efeats,
        w['w1me_p'], w['w2me_p'], w['w2ee_p'], w['w1ee_p'],
        w['w1me_q'], w['w2me_q'], w['w2ee_q'], w['w1ee_q'])
    t1_p, t1_q, c1_p, c1_q = _node_a(
        player_x, match_x,
        w['w1ms_p'], w['b1m_p'], w['w1es_p'], w['b1e_p'],
        w['w1ms_q'], w['b1m_q'], w['w1es_q'], w['b1e_q'])

    counts = _sc_counts(jnp.stack([p_dst, q_dst]))

    s1c_p = _p1(t1_p, b1_p, p_src, p_dst)
    s1c_q = _p1(t1_q, b1_q, q_src, q_dst)

    (mh1, ph1, ts2_p, td2_p, ts2_q, td2_q,
     hn1_p, hn1_q, cnt_p, cnt_q) = _stage_b(
        player_x, match_x, s1c_p, s1c_q, counts, c1_p, c1_q, w)

    s2_p = _p2(ts2_p, td2_p, eB_p, p_src, p_dst)
    s2_q = _p2(ts2_q, td2_q, eB_q, q_src, q_dst)

    mout, pout, us_p, ud_p, us_q, ud_q = _stage_c(
        s2_p, s2_q, cnt_p, cnt_q, mh1, ph1, hn1_p, hn1_q, c1_p, c1_q, w)

    plays_eh2 = _p3(us_p, ud_p, eD_p, p_src, p_dst)
    pb_eh2 = _p3(us_q, ud_q, eD_q, q_src, q_dst)

    return (mout, pout, plays_eh2, pb_eh2)
